# baseline (speedup 1.0000x reference)
import functools

import jax
import jax.numpy as jnp
from jax import lax
from jax.experimental import pallas as pl
from jax.experimental.pallas import tpu as pltpu
from jax.experimental.pallas import tpu_sc as plsc

_NUM_CORES = 2
_NUM_SUBCORES = 16
_NUM_WORKERS = _NUM_CORES * _NUM_SUBCORES


def kernel(tokens, W_pos):
    batch, seq_len = tokens.shape
    n_ctx, d_model = W_pos.shape
    rows_per_w = seq_len // _NUM_WORKERS

    mesh = plsc.VectorSubcoreMesh(
        core_axis_name="c",
        subcore_axis_name="s",
        num_cores=_NUM_CORES,
        num_subcores=_NUM_SUBCORES,
    )

    @functools.partial(
        pl.kernel,
        out_type=jax.ShapeDtypeStruct((batch, seq_len, d_model), W_pos.dtype),
        mesh=mesh,
        scratch_types=[
            pltpu.VMEM((rows_per_w, d_model), W_pos.dtype),
            pltpu.SemaphoreType.DMA,
        ],
    )
    def body(w_hbm, out_hbm, buf, sem):
        wid = lax.axis_index("s") * _NUM_CORES + lax.axis_index("c")
        base = wid * rows_per_w
        pltpu.sync_copy(w_hbm.at[pl.ds(base, rows_per_w), :], buf)
        copies = [
            pltpu.async_copy(buf, out_hbm.at[b, pl.ds(base, rows_per_w), :], sem)
            for b in range(batch)
        ]
        for c in copies:
            c.wait()

    return body(W_pos)


# speedup vs baseline: 2.0716x; 2.0716x over previous
import functools

import jax
import jax.numpy as jnp
from jax import lax
from jax.experimental import pallas as pl
from jax.experimental.pallas import tpu as pltpu
from jax.experimental.pallas import tpu_sc as plsc

_NUM_CORES = 2
_NUM_SUBCORES = 16
_NUM_WORKERS = _NUM_CORES * _NUM_SUBCORES


def kernel(tokens, W_pos):
    batch, seq_len = tokens.shape
    n_ctx, d_model = W_pos.shape

    def body(w_ref, out_ref):
        out_ref[...] = w_ref[...][None]

    return pl.pallas_call(
        body,
        grid=(batch,),
        in_specs=[pl.BlockSpec((seq_len, d_model), lambda b: (0, 0))],
        out_specs=pl.BlockSpec((1, seq_len, d_model), lambda b: (b, 0, 0)),
        out_shape=jax.ShapeDtypeStruct((batch, seq_len, d_model), W_pos.dtype),
    )(W_pos)


def _kernel_sc(tokens, W_pos):
    batch, seq_len = tokens.shape
    n_ctx, d_model = W_pos.shape
    rows_per_w = seq_len // _NUM_WORKERS

    mesh = plsc.VectorSubcoreMesh(
        core_axis_name="c",
        subcore_axis_name="s",
        num_cores=_NUM_CORES,
        num_subcores=_NUM_SUBCORES,
    )

    @functools.partial(
        pl.kernel,
        out_type=jax.ShapeDtypeStruct((batch, seq_len, d_model), W_pos.dtype),
        mesh=mesh,
        scratch_types=[
            pltpu.VMEM((rows_per_w, d_model), W_pos.dtype),
            pltpu.SemaphoreType.DMA,
        ],
    )
    def body(w_hbm, out_hbm, buf, sem):
        wid = lax.axis_index("s") * _NUM_CORES + lax.axis_index("c")
        base = wid * rows_per_w
        pltpu.sync_copy(w_hbm.at[pl.ds(base, rows_per_w), :], buf)
        copies = [
            pltpu.async_copy(buf, out_hbm.at[b, pl.ds(base, rows_per_w), :], sem)
            for b in range(batch)
        ]
        for c in copies:
            c.wait()

    return body(W_pos)
